# baseline (speedup 1.0000x reference)
import functools

import jax
import jax.numpy as jnp
from jax import lax
from jax.experimental import pallas as pl
from jax.experimental.pallas import tpu as pltpu
from jax.experimental.pallas import tpu_sc as plsc

L = 16
NC = 2
NS = 16
NW = NC * NS

VOCAB = 2000
DIM = 8
B = 16384
S = 50
BT = B // 128

ROWS_W = B // NW
ROWS_C = 128
N_CHUNKS = ROWS_W // ROWS_C
CHUNK = ROWS_C * S


@functools.partial(
    pl.kernel,
    out_type=jax.ShapeDtypeStruct((S, BT, DIM, 128), jnp.float32),
    mesh=plsc.VectorSubcoreMesh(core_axis_name="c", subcore_axis_name="s"),
    compiler_params=pltpu.CompilerParams(needs_layout_passes=False),
    scratch_types=[
        pltpu.VMEM((VOCAB * DIM,), jnp.float32),
        pltpu.VMEM((CHUNK,), jnp.int32),
        pltpu.VMEM((S, 1, DIM, 128), jnp.float32),
        pltpu.VMEM((S, 1, DIM, 128), jnp.float32),
        pltpu.SemaphoreType.DMA,
        pltpu.SemaphoreType.DMA,
        pltpu.SemaphoreType.DMA,
    ],
)
def _emb_lookup(ids_hbm, table_hbm, out_hbm, table_v, idx_v,
                out_v0, out_v1, sem_t, sem_o0, sem_o1):
    wid = lax.axis_index("s") * NC + lax.axis_index("c")
    tbl_h = pltpu.async_copy(table_hbm, table_v, sem_t)

    iota_s = lax.iota(jnp.int32, L) * S
    out_bufs = (out_v0, out_v1)
    out_sems = (sem_o0, sem_o1)
    out_handles = [None, None]

    NG = ROWS_C // L

    def run_s_loop(out_v):
        @plsc.parallel_loop(0, S, 1, unroll=1)
        def s_body(s):
            idsv = [plsc.load_gather(idx_v, [iota_s + (g * L * S) + s])
                    for g in range(NG)]
            rbs = [v * DIM for v in idsv]
            cols = [[plsc.load_gather(table_v, [rbs[g] + j])
                     for j in range(DIM)] for g in range(NG)]
            for g in range(NG):
                for j in range(DIM):
                    out_v[s, 0, j, pl.ds(g * L, L)] = cols[g][j]

    for c in range(N_CHUNKS):
        bt = wid * N_CHUNKS + c
        buf = c % 2
        pltpu.sync_copy(ids_hbm.at[pl.ds(bt * CHUNK, CHUNK)], idx_v)
        if c == 0:
            tbl_h.wait()
        if out_handles[buf] is not None:
            out_handles[buf].wait()
        run_s_loop(out_bufs[buf])
        out_handles[buf] = pltpu.async_copy(
            out_bufs[buf], out_hbm.at[:, pl.ds(bt, 1)], out_sems[buf])

    out_handles[0].wait()
    out_handles[1].wait()


def kernel(ids, base_embedding):
    ids_flat = ids.reshape(-1).astype(jnp.int32)
    table_flat = base_embedding.reshape(-1).astype(jnp.float32)
    out4 = _emb_lookup(ids_flat, table_flat)
    return out4.transpose(1, 3, 0, 2).reshape(B, S, DIM)


# speedup vs baseline: 1.1757x; 1.1757x over previous
import functools

import jax
import jax.numpy as jnp
from jax import lax
from jax.experimental import pallas as pl
from jax.experimental.pallas import tpu as pltpu
from jax.experimental.pallas import tpu_sc as plsc

L = 16
NC = 2
NS = 16
NW = NC * NS

VOCAB = 2000
DIM = 8
B = 16384
S = 50
SH = S // 2
BT = B // 128

ROWS_W = B // NW
ROWS_C = 128
N_CHUNKS = ROWS_W // ROWS_C
CHUNK = ROWS_C * S
NG = ROWS_C // L


@functools.partial(
    pl.kernel,
    out_type=jax.ShapeDtypeStruct((S, BT, DIM, 128), jnp.float32),
    mesh=plsc.VectorSubcoreMesh(core_axis_name="c", subcore_axis_name="s"),
    compiler_params=pltpu.CompilerParams(needs_layout_passes=False),
    scratch_types=[
        pltpu.VMEM((VOCAB * DIM,), jnp.float32),
        pltpu.VMEM((CHUNK,), jnp.int32),
        pltpu.VMEM((CHUNK,), jnp.int32),
        pltpu.VMEM((SH, 1, DIM, 128), jnp.float32),
        pltpu.VMEM((SH, 1, DIM, 128), jnp.float32),
        pltpu.SemaphoreType.DMA,
        pltpu.SemaphoreType.DMA,
        pltpu.SemaphoreType.DMA,
        pltpu.SemaphoreType.DMA,
        pltpu.SemaphoreType.DMA,
    ],
)
def _emb_lookup(ids_hbm, table_hbm, out_hbm, table_v, idx_v0, idx_v1,
                out_v0, out_v1, sem_t, sem_i0, sem_i1, sem_o0, sem_o1):
    wid = lax.axis_index("s") * NC + lax.axis_index("c")
    bt0 = wid * N_CHUNKS
    tbl_h = pltpu.async_copy(table_hbm, table_v, sem_t)

    idx_bufs = (idx_v0, idx_v1)
    idx_sems = (sem_i0, sem_i1)
    out_bufs = (out_v0, out_v1)
    out_sems = (sem_o0, sem_o1)
    out_handles = [None, None]

    giotas = [lax.iota(jnp.int32, L) * S + (g * L * S) for g in range(NG)]

    idx_handles = [
        pltpu.async_copy(ids_hbm.at[pl.ds(bt0 * CHUNK, CHUNK)], idx_v0, sem_i0),
        None,
    ]

    def make_s_body(idx_v, out_v, s_off):
        def load_group(s, g):
            ids16 = plsc.load_gather(idx_v, [giotas[g] + s])
            rb = ids16 * DIM
            return [plsc.load_gather(table_v, [rb + j]) for j in range(DIM)]

        def store_group(sl, g, cols):
            for j in range(DIM):
                out_v[sl, 0, j, pl.ds(g * L, L)] = cols[j]

        def s_body(sl, carry2):
            s = sl + s_off
            cols = load_group(s, 0)
            for g in range(1, NG):
                nxt = load_group(s, g)
                store_group(sl, g - 1, cols)
                cols = nxt
            store_group(sl, NG - 1, cols)
            return carry2
        return s_body

    for c in range(N_CHUNKS):
        bt = bt0 + c
        buf = c % 2
        idx_handles[buf].wait()
        if c == 0:
            tbl_h.wait()
        if c + 1 < N_CHUNKS:
            nbuf = (c + 1) % 2
            idx_handles[nbuf] = pltpu.async_copy(
                ids_hbm.at[pl.ds((bt + 1) * CHUNK, CHUNK)],
                idx_bufs[nbuf], idx_sems[nbuf])
        for h in range(2):
            ob = out_bufs[h]
            if out_handles[h] is not None:
                out_handles[h].wait()
            lax.fori_loop(0, SH, make_s_body(idx_bufs[buf], ob, h * SH), 0)
            out_handles[h] = pltpu.async_copy(
                ob, out_hbm.at[pl.ds(h * SH, SH), pl.ds(bt, 1)], out_sems[h])

    out_handles[0].wait()
    out_handles[1].wait()


def kernel(ids, base_embedding):
    ids_flat = ids.reshape(-1).astype(jnp.int32)
    table_flat = base_embedding.reshape(-1).astype(jnp.float32)
    out4 = _emb_lookup(ids_flat, table_flat)
    return out4.transpose(1, 3, 0, 2).reshape(B, S, DIM)


# speedup vs baseline: 1.4257x; 1.2127x over previous
import functools

import jax
import jax.numpy as jnp
from jax import lax
from jax.experimental import pallas as pl
from jax.experimental.pallas import tpu as pltpu
from jax.experimental.pallas import tpu_sc as plsc

L = 16
NC = 2
NS = 16
NW = NC * NS

VOCAB = 2000
DIM = 8
B = 16384
S = 50
SH = S // 2
BT = B // 128

ROWS_W = B // NW
ROWS_C = 128
N_CHUNKS = ROWS_W // ROWS_C
CHUNK = ROWS_C * S
NG = ROWS_C // L


@functools.partial(
    pl.kernel,
    out_type=jax.ShapeDtypeStruct((S, BT, DIM, 128), jnp.float32),
    mesh=plsc.VectorSubcoreMesh(core_axis_name="c", subcore_axis_name="s"),
    compiler_params=pltpu.CompilerParams(needs_layout_passes=False),
    scratch_types=[
        pltpu.VMEM((VOCAB * DIM,), jnp.float32),
        pltpu.VMEM((CHUNK,), jnp.int32),
        pltpu.VMEM((CHUNK,), jnp.int32),
        pltpu.VMEM((SH, 1, DIM, 128), jnp.float32),
        pltpu.VMEM((SH, 1, DIM, 128), jnp.float32),
        pltpu.SemaphoreType.DMA,
        pltpu.SemaphoreType.DMA,
        pltpu.SemaphoreType.DMA,
        pltpu.SemaphoreType.DMA,
        pltpu.SemaphoreType.DMA,
    ],
)
def _emb_lookup(ids_hbm, table_hbm, out_hbm, table_v, idx_v0, idx_v1,
                out_v0, out_v1, sem_t, sem_i0, sem_i1, sem_o0, sem_o1):
    wid = lax.axis_index("s") * NC + lax.axis_index("c")
    bt0 = wid * N_CHUNKS
    tbl_h = pltpu.async_copy(table_hbm, table_v, sem_t)

    idx_bufs = (idx_v0, idx_v1)
    idx_sems = (sem_i0, sem_i1)
    out_bufs = (out_v0, out_v1)
    out_sems = (sem_o0, sem_o1)
    out_handles = [None, None]

    giotas = [lax.iota(jnp.int32, L) * S + (g * L * S) for g in range(NG)]

    idx_handles = [
        pltpu.async_copy(ids_hbm.at[pl.ds(bt0 * CHUNK, CHUNK)], idx_v0, sem_i0),
        None,
    ]

    def make_s_body(idx_v, out_v, s_off):
        def s_body(sl, carry2):
            s = sl + s_off
            rbs = [plsc.load_gather(idx_v, [giotas[g] + s]) * DIM
                   for g in range(NG)]
            cols = [plsc.load_gather(table_v, [rbs[0] + j])
                    for j in range(DIM)]
            for g in range(1, NG):
                nxt = []
                for j in range(DIM):
                    nxt.append(plsc.load_gather(table_v, [rbs[g] + j]))
                    out_v[sl, 0, j, pl.ds((g - 1) * L, L)] = cols[j]
                cols = nxt
            for j in range(DIM):
                out_v[sl, 0, j, pl.ds((NG - 1) * L, L)] = cols[j]
            return carry2
        return s_body

    for c in range(N_CHUNKS):
        bt = bt0 + c
        buf = c % 2
        idx_handles[buf].wait()
        if c == 0:
            tbl_h.wait()
        if c + 1 < N_CHUNKS:
            nbuf = (c + 1) % 2
            idx_handles[nbuf] = pltpu.async_copy(
                ids_hbm.at[pl.ds((bt + 1) * CHUNK, CHUNK)],
                idx_bufs[nbuf], idx_sems[nbuf])
        for h in range(2):
            ob = out_bufs[h]
            if out_handles[h] is not None:
                out_handles[h].wait()
            lax.fori_loop(0, SH, make_s_body(idx_bufs[buf], ob, h * SH), 0)
            out_handles[h] = pltpu.async_copy(
                ob, out_hbm.at[pl.ds(h * SH, SH), pl.ds(bt, 1)], out_sems[h])

    out_handles[0].wait()
    out_handles[1].wait()


def kernel(ids, base_embedding):
    ids_flat = ids.reshape(-1).astype(jnp.int32)
    table_flat = base_embedding.reshape(-1).astype(jnp.float32)
    out4 = _emb_lookup(ids_flat, table_flat)
    return out4.transpose(1, 3, 0, 2).reshape(B, S, DIM)


# speedup vs baseline: 1.4348x; 1.0064x over previous
import functools

import jax
import jax.numpy as jnp
from jax import lax
from jax.experimental import pallas as pl
from jax.experimental.pallas import tpu as pltpu
from jax.experimental.pallas import tpu_sc as plsc

L = 16
NC = 2
NS = 16
NW = NC * NS

VOCAB = 2000
DIM = 8
B = 16384
S = 50
SH = S // 2
BT = B // 128

ROWS_W = B // NW
ROWS_C = 128
N_CHUNKS = ROWS_W // ROWS_C
CHUNK = ROWS_C * S
NG = ROWS_C // L


@functools.partial(
    pl.kernel,
    out_type=jax.ShapeDtypeStruct((S, BT, DIM, 128), jnp.float32),
    mesh=plsc.VectorSubcoreMesh(core_axis_name="c", subcore_axis_name="s"),
    compiler_params=pltpu.CompilerParams(needs_layout_passes=False),
    scratch_types=[
        pltpu.VMEM((VOCAB * DIM,), jnp.float32),
        pltpu.VMEM((CHUNK,), jnp.int32),
        pltpu.VMEM((CHUNK,), jnp.int32),
        pltpu.VMEM((SH, 1, DIM, 128), jnp.float32),
        pltpu.VMEM((SH, 1, DIM, 128), jnp.float32),
        pltpu.SemaphoreType.DMA,
        pltpu.SemaphoreType.DMA,
        pltpu.SemaphoreType.DMA,
        pltpu.SemaphoreType.DMA,
        pltpu.SemaphoreType.DMA,
    ],
)
def _emb_lookup(ids_hbm, table_hbm, out_hbm, table_v, idx_v0, idx_v1,
                out_v0, out_v1, sem_t, sem_i0, sem_i1, sem_o0, sem_o1):
    wid = lax.axis_index("s") * NC + lax.axis_index("c")
    bt0 = wid * N_CHUNKS
    NP = 8
    PIECE = VOCAB * DIM // NP
    tbl_handles = []
    for k in range(NP):
        off = ((wid + k) % NP) * PIECE
        tbl_handles.append(pltpu.async_copy(
            table_hbm.at[pl.ds(off, PIECE)], table_v.at[pl.ds(off, PIECE)],
            sem_t))

    idx_bufs = (idx_v0, idx_v1)
    idx_sems = (sem_i0, sem_i1)
    out_bufs = (out_v0, out_v1)
    out_sems = (sem_o0, sem_o1)
    out_handles = [None, None]

    giotas = [lax.iota(jnp.int32, L) * S + (g * L * S) for g in range(NG)]

    idx_handles = [
        pltpu.async_copy(ids_hbm.at[pl.ds(bt0 * CHUNK, CHUNK)], idx_v0, sem_i0),
        None,
    ]

    def make_s_body(idx_v, out_v, s_off):
        def s_body(sl, carry2):
            s = sl + s_off
            rbs = [plsc.load_gather(idx_v, [giotas[g] + s]) * DIM
                   for g in range(NG)]
            cols = [plsc.load_gather(table_v, [rbs[0] + j])
                    for j in range(DIM)]
            for g in range(1, NG):
                nxt = []
                for j in range(DIM):
                    nxt.append(plsc.load_gather(table_v, [rbs[g] + j]))
                    out_v[sl, 0, j, pl.ds((g - 1) * L, L)] = cols[j]
                cols = nxt
            for j in range(DIM):
                out_v[sl, 0, j, pl.ds((NG - 1) * L, L)] = cols[j]
            return carry2
        return s_body

    for c in range(N_CHUNKS):
        bt = bt0 + c
        buf = c % 2
        idx_handles[buf].wait()
        if c == 0:
            for h in tbl_handles:
                h.wait()
        if c + 1 < N_CHUNKS:
            nbuf = (c + 1) % 2
            idx_handles[nbuf] = pltpu.async_copy(
                ids_hbm.at[pl.ds((bt + 1) * CHUNK, CHUNK)],
                idx_bufs[nbuf], idx_sems[nbuf])
        for h in range(2):
            ob = out_bufs[h]
            if out_handles[h] is not None:
                out_handles[h].wait()
            lax.fori_loop(0, SH, make_s_body(idx_bufs[buf], ob, h * SH), 0)
            out_handles[h] = pltpu.async_copy(
                ob, out_hbm.at[pl.ds(h * SH, SH), pl.ds(bt, 1)], out_sems[h])

    out_handles[0].wait()
    out_handles[1].wait()


def kernel(ids, base_embedding):
    ids_flat = ids.reshape(-1).astype(jnp.int32)
    table_flat = base_embedding.reshape(-1).astype(jnp.float32)
    out4 = _emb_lookup(ids_flat, table_flat)
    return out4.transpose(1, 3, 0, 2).reshape(B, S, DIM)
